# baseline (speedup 1.0000x reference)
import functools

import jax
import jax.numpy as jnp
from jax import lax
from jax.experimental import pallas as pl
from jax.experimental.pallas import tpu as pltpu
from jax.experimental.pallas import tpu_sc as plsc

_NC, _NS, _L = 2, 16, 16
_NW = _NC * _NS


def _diag_sc_body(x_hbm, out_hbm, blk_v, out_v, sem0, sem1):
    wid = lax.axis_index("s") * _NC + lax.axis_index("c")
    mats_per_w = out_v.shape[0]
    m = x_hbm.shape[1]
    nq = m // 128
    total = mats_per_w * nq
    sems = [sem0, sem1]
    neg = jnp.float32(-jnp.inf)
    iota = lax.iota(jnp.int32, _L)

    def start(i):
        mat = wid * mats_per_w + i // nq
        q = i % nq
        return pltpu.async_copy(
            x_hbm.at[mat, pl.ds(128 * q, 128), pl.ds(128 * q, 128)],
            blk_v.at[i % 2],
            sems[i % 2],
        )

    cp = start(0)
    acc = jnp.full((_L,), neg, dtype=jnp.float32)
    for i in range(total):
        cp.wait()
        if i + 1 < total:
            nxt = start(i + 1)
        buf = i % 2
        for k in range(8):
            def body(l, a, k=k, buf=buf):
                v = blk_v[buf, 16 * k + l, pl.ds(16 * k, _L)]
                return jnp.maximum(a, jnp.where(iota == l, v, neg))

            acc = lax.fori_loop(0, _L, body, acc)
        if i % nq == nq - 1:
            out_v[i // nq] = acc
            acc = jnp.full((_L,), neg, dtype=jnp.float32)
        if i + 1 < total:
            cp = nxt
    pltpu.sync_copy(out_v, out_hbm.at[wid])


def _sc_diag_max(x3d):
    n_mat, m, _ = x3d.shape
    mats_per_w = n_mat // _NW
    mesh = plsc.VectorSubcoreMesh(core_axis_name="c", subcore_axis_name="s")
    return functools.partial(
        pl.kernel,
        mesh=mesh,
        out_type=jax.ShapeDtypeStruct((_NW, mats_per_w, _L), jnp.float32),
        scratch_types=[
            pltpu.VMEM((2, 128, 128), jnp.float32),
            pltpu.VMEM((mats_per_w, _L), jnp.float32),
            pltpu.SemaphoreType.DMA,
            pltpu.SemaphoreType.DMA,
        ],
        compiler_params=pltpu.CompilerParams(use_tc_tiling_on_sc=True),
    )(_diag_sc_body)(x3d)


def _maxes_body(x_ref, out_ref):
    i = pl.program_id(0)
    x = x_ref[...]
    N, m, _ = x.shape
    C2 = out_ref.shape[1]
    C = C2 // 2
    per_row = C // N
    row = jax.lax.broadcasted_iota(jnp.int32, (m, m), 0)
    col = jax.lax.broadcasted_iota(jnp.int32, (m, m), 1)
    eq = (row == col)[None]
    neg = jnp.float32(-jnp.inf)
    dmax = jnp.max(jnp.where(eq, x, neg), axis=(1, 2)).reshape(1, N)
    omax = jnp.max(jnp.where(eq, neg, x), axis=(1, 2)).reshape(1, N)
    n_steps = pl.num_programs(0)
    for step in range(n_steps):
        b = step // per_row
        c0 = (step % per_row) * N

        @pl.when(i == step)
        def _(b=b, c0=c0):
            out_ref[b : b + 1, c0 : c0 + N] = dmax
            out_ref[b : b + 1, C + c0 : C + c0 + N] = omax


def kernel(x):
    B, C, m, _ = x.shape
    n_mat = B * C
    x3d = x.reshape(n_mat, m, m)
    sc_out = _sc_diag_max(x3d)
    diag = jnp.max(sc_out.reshape(n_mat, _L), axis=-1).reshape(B, C)
    N = 8
    tc_out = pl.pallas_call(
        _maxes_body,
        grid=(n_mat // N,),
        in_specs=[pl.BlockSpec((N, m, m), lambda i: (i, 0, 0))],
        out_specs=pl.BlockSpec((B, 2 * C), lambda i: (0, 0)),
        out_shape=jax.ShapeDtypeStruct((B, 2 * C), x.dtype),
    )(x3d)
    return jnp.concatenate((diag, tc_out[:, C:]), axis=-1)


# speedup vs baseline: 1.7128x; 1.7128x over previous
import jax
import jax.numpy as jnp
from jax.experimental import pallas as pl


def _maxes_body(x_ref, out_ref):
    i = pl.program_id(0)
    x = x_ref[...]
    N, m, _ = x.shape
    C2 = out_ref.shape[1]
    C = C2 // 2
    per_row = C // N
    row = jax.lax.broadcasted_iota(jnp.int32, (m, m), 0)
    col = jax.lax.broadcasted_iota(jnp.int32, (m, m), 1)
    eq = (row == col)[None]
    neg = jnp.float32(-jnp.inf)
    dmax = jnp.max(jnp.where(eq, x, neg), axis=(1, 2)).reshape(1, N)
    omax = jnp.max(jnp.where(eq, neg, x), axis=(1, 2)).reshape(1, N)
    n_steps = pl.num_programs(0)
    for step in range(n_steps):
        b = step // per_row
        c0 = (step % per_row) * N

        @pl.when(i == step)
        def _(b=b, c0=c0):
            out_ref[b : b + 1, c0 : c0 + N] = dmax
            out_ref[b : b + 1, C + c0 : C + c0 + N] = omax


def kernel(x):
    B, C, m, _ = x.shape
    n_mat = B * C
    N = 8
    return pl.pallas_call(
        _maxes_body,
        grid=(n_mat // N,),
        in_specs=[pl.BlockSpec((N, m, m), lambda i: (i, 0, 0))],
        out_specs=pl.BlockSpec((B, 2 * C), lambda i: (0, 0)),
        out_shape=jax.ShapeDtypeStruct((B, 2 * C), x.dtype),
    )(x.reshape(n_mat, m, m))
